# Initial kernel scaffold; baseline (speedup 1.0000x reference)
import jax
import jax.numpy as jnp
from jax.experimental import pallas as pl


def kernel(audio, bias_spec):
    raise NotImplementedError("write your pallas kernel here")


# speedup vs baseline: 1.6867x; 1.6867x over previous
import functools

import numpy as np
import jax
import jax.numpy as jnp
from jax.experimental import pallas as pl
from jax.experimental.pallas import tpu as pltpu

_FL = 64
_HOP = _FL // 4
_CUT = _FL // 2 + 1
_LANES = 128
_HALF = _LANES // 2
_STRENGTH = 0.1


def _stft_bases():
    four = np.fft.fft(np.eye(_FL))
    fb = np.vstack([np.real(four[:_CUT]), np.imag(four[:_CUT])])
    n = np.arange(_FL)
    win = 0.5 - 0.5 * np.cos(2.0 * np.pi * n / _FL)
    fwd = (fb * win[None, :]).astype(np.float32)
    inv = (np.linalg.pinv((_FL / _HOP) * fb).T * win[None, :]).astype(np.float32)
    return fwd, inv, win.astype(np.float32)


def _phase_consts():
    fwd, inv, _ = _stft_bases()
    fpack = np.zeros((_FL, _LANES), np.float32)
    fpack[:, :_CUT] = fwd[:_CUT].T
    fpack[:, _HALF:_HALF + _CUT] = fwd[_CUT:].T
    ipack = np.zeros((_LANES, _LANES), np.float32)
    ipack[:_CUT, :_FL] = inv[:_CUT]
    ipack[_HALF:_HALF + _CUT, :_FL] = inv[_CUT:]
    cfwd = np.zeros((4 * _LANES, _LANES), np.float32)
    cinv = np.zeros((4 * _LANES, _LANES), np.float32)
    for i in range(4):
        cfwd[128 * i + _HOP * i:128 * i + _HOP * i + _FL, :] = fpack
        cinv[128 * i:128 * (i + 1), :] = np.roll(ipack, _HOP * i, axis=1)
    return cfwd, cinv


_CFWD, _CINV = _phase_consts()


def _inv_window(n_frames):
    _, _, win = _stft_bases()
    win_sq = win.astype(np.float64) ** 2
    out_len = _FL + _HOP * (n_frames - 1)
    wss = np.zeros(out_len, np.float64)
    for i in range(n_frames):
        wss[i * _HOP:i * _HOP + _FL] += win_sq
    tiny = np.finfo(np.float32).tiny
    scale = _FL / _HOP
    return np.where(wss > tiny, scale / wss, scale).astype(np.float32)


def _body(x_ref, bias_ref, cfwd_ref, cinv_ref, invwin_ref, out_ref, *, nr, gmax):
    x = x_ref[0]
    xs = pltpu.roll(x, shift=nr - 1, axis=0)
    w = jnp.concatenate([x, xs], axis=1)
    bias = bias_ref[0:1, :]
    remask = bias_ref[1:2, :]
    row_id = jax.lax.broadcasted_iota(jnp.int32, (nr, 1), 0)

    acc = None
    for i in range(4):
        spec = jnp.dot(w, cfwd_ref[128 * i:128 * (i + 1), :],
                       preferred_element_type=jnp.float32)
        spec = jnp.where(row_id <= gmax[i], spec, 0.0)
        sq = spec * spec
        mag = jnp.sqrt(sq + pltpu.roll(sq, shift=_HALF, axis=1))
        nz = mag > 0.0
        safe = jnp.where(nz, mag, 1.0)
        den = jnp.maximum(mag - bias, 0.0)
        ratio = den * pl.reciprocal(safe, approx=True)
        new_spec = jnp.where(nz, spec * ratio, den * remask)
        contrib = jnp.dot(new_spec, cinv_ref[128 * i:128 * (i + 1), :],
                          preferred_element_type=jnp.float32)
        acc = contrib if acc is None else acc + contrib

    spill = pltpu.roll(acc[:, _FL:], shift=1, axis=0)
    out_ref[0] = (acc[:, :_FL] + spill) * invwin_ref[...]


def kernel(audio, bias_spec):
    audio = audio.astype(jnp.float32)
    B, T = audio.shape
    assert T % _HOP == 0
    pad = _FL // 2
    padded = jnp.pad(audio, ((0, 0), (pad, pad)), mode="reflect")
    P = T + _FL
    n_frames = T // _HOP + 1
    nr = -(-(P // _FL + 1) // 8) * 8
    xall = jnp.pad(padded, ((0, 0), (0, nr * _FL - P)))
    x3 = xall.reshape(B, nr, _FL)

    bias_vec = bias_spec[0, :, 0].astype(jnp.float32) * _STRENGTH
    small = jnp.zeros((2, _LANES), jnp.float32)
    small = small.at[0, :_CUT].set(bias_vec)
    small = small.at[0, _HALF:_HALF + _CUT].set(bias_vec)
    small = small.at[1, :_CUT].set(1.0)

    invwin = np.ones(nr * _FL, np.float32)
    invwin[:P] = np.pad(_inv_window(n_frames), (0, P - (_FL + _HOP * (n_frames - 1))))
    invwin = jnp.asarray(invwin.reshape(nr, _FL))

    gmax = tuple((n_frames - 1 - i) // 4 for i in range(4))
    body = functools.partial(_body, nr=nr, gmax=gmax)

    out = pl.pallas_call(
        body,
        out_shape=jax.ShapeDtypeStruct((B, nr, _FL), jnp.float32),
        grid=(B,),
        in_specs=[
            pl.BlockSpec((1, nr, _FL), lambda b: (b, 0, 0)),
            pl.BlockSpec((2, _LANES), lambda b: (0, 0)),
            pl.BlockSpec((4 * _LANES, _LANES), lambda b: (0, 0)),
            pl.BlockSpec((4 * _LANES, _LANES), lambda b: (0, 0)),
            pl.BlockSpec((nr, _FL), lambda b: (0, 0)),
        ],
        out_specs=pl.BlockSpec((1, nr, _FL), lambda b: (b, 0, 0)),
        compiler_params=pltpu.CompilerParams(dimension_semantics=("parallel",)),
    )(x3, small, jnp.asarray(_CFWD), jnp.asarray(_CINV), invwin)

    full = out.reshape(B, nr * _FL)
    return full[:, pad:pad + T][:, None, :]
